# baseline (speedup 1.0000x reference)
import jax
import jax.numpy as jnp
from jax import lax
from jax.experimental import pallas as pl
from jax.experimental.pallas import tpu as pltpu
from jax.experimental.pallas import tpu_sc as plsc

_F = 100
_D = 32
_BATCH = 16384
_ROW = _F * _D
_LANES = 16

_NC = 2
_NS = 16
_NW = _NC * _NS
_BPW = _BATCH // _NW

_FC = 2
_BC = 128
_NBC = _BPW // _BC
_NCH = (_F // _FC) * _NBC
_KB = _BC // _LANES
_TS = _FC * _D * _LANES


def _sc_body(x_hbm, tab_hbm, out_hbm, tab_v, bi, bo,
             si0, si1, si2, so0, so1, so2):
    sins = (si0, si1, si2)
    souts = (so0, so1, so2)
    wid = lax.axis_index("s") * _NC + lax.axis_index("c")
    col0 = wid * _BPW
    pltpu.sync_copy(tab_hbm, tab_v)

    def offs(g):
        f0 = (g // _NBC) * _FC
        b0 = col0 + (g % _NBC) * _BC
        return f0, b0

    def start_in(g, s):
        f0, b0 = offs(g)
        pltpu.async_copy(
            x_hbm.at[pl.ds(f0, _FC), :, pl.ds(b0, _BC)], bi.at[s], sins[s]
        )

    def wait_in(s):
        pltpu.make_async_copy(
            x_hbm.at[pl.ds(0, _FC), :, pl.ds(0, _BC)], bi.at[s], sins[s]
        ).wait()

    def start_out(g, s):
        f0, b0 = offs(g)
        pltpu.async_copy(
            bo.at[s], out_hbm.at[pl.ds(f0, _FC), :, pl.ds(b0, _BC)], souts[s]
        )

    def wait_out(s):
        pltpu.make_async_copy(
            bo.at[s], out_hbm.at[pl.ds(0, _FC), :, pl.ds(0, _BC)], souts[s]
        ).wait()

    def compute(g, s):
        f0 = (g // _NBC) * _FC
        for f in range(_FC):
            def dbody(d, c2):
                t = tab_v[0, pl.ds(((f0 + f) * _D + d) * _LANES, _LANES)]
                for k in range(_KB):
                    sl = pl.ds(k * _LANES, _LANES)
                    bo[s, f, d, sl] = bi[s, f, d, sl] + t
                return c2

            lax.fori_loop(0, _D, dbody, 0, unroll=2)

    def step(g, s, with_wait_out):
        wait_in(s)
        nxt = jnp.minimum(g + 2, _NCH - 1)
        start_in(nxt, (s + 2) % 3)
        if with_wait_out:
            wait_out(s)
        compute(g, s)
        start_out(g, s)

    start_in(0, 0)
    start_in(1, 1)
    for g in range(3):
        step(g, g, with_wait_out=False)

    def trip(p, carry):
        for j in range(3):
            step(3 + p * 3 + j, j, with_wait_out=True)
        return carry

    lax.fori_loop(0, (_NCH - 5) // 3, trip, 0)
    step(_NCH - 2, (_NCH - 2) % 3, with_wait_out=True)
    step(_NCH - 1, (_NCH - 1) % 3, with_wait_out=True)

    wait_in((_NCH) % 3)
    wait_in((_NCH + 1) % 3)
    wait_out((_NCH - 3) % 3)
    wait_out((_NCH - 2) % 3)
    wait_out((_NCH - 1) % 3)


def kernel(inputs, column_table):
    xt = jnp.transpose(inputs, (1, 2, 0))
    tab = jnp.repeat(column_table.reshape(-1), _LANES).reshape(1, _ROW * _LANES)
    mesh = plsc.VectorSubcoreMesh(core_axis_name="c", subcore_axis_name="s")
    out_t = pl.kernel(
        _sc_body,
        out_type=jax.ShapeDtypeStruct((_F, _D, _BATCH), jnp.float32),
        mesh=mesh,
        scratch_types=[
            pltpu.VMEM((1, _ROW * _LANES), jnp.float32),
            pltpu.VMEM((3, _FC, _D, _BC), jnp.float32),
            pltpu.VMEM((3, _FC, _D, _BC), jnp.float32),
            pltpu.SemaphoreType.DMA,
            pltpu.SemaphoreType.DMA,
            pltpu.SemaphoreType.DMA,
            pltpu.SemaphoreType.DMA,
            pltpu.SemaphoreType.DMA,
            pltpu.SemaphoreType.DMA,
        ],
    )(xt, tab)
    return jnp.transpose(out_t, (2, 0, 1))


# speedup vs baseline: 2.2563x; 2.2563x over previous
import jax
import jax.numpy as jnp
from jax import lax
from jax.experimental import pallas as pl
from jax.experimental.pallas import tpu as pltpu
from jax.experimental.pallas import tpu_sc as plsc

_F = 100
_D = 32
_BATCH = 16384
_ROW = _F * _D
_LANES = 16

_NC = 2
_NS = 16
_NW = _NC * _NS
_BPW = _BATCH // _NW

_FC = 2
_BC = 256
_NBC = _BPW // _BC
_NCH = (_F // _FC) * _NBC
_KB = _BC // _LANES


def _sc_body(x_hbm, tab_hbm, out_hbm, tab_v, bb,
             si0, si1, si2, si3, so0, so1, so2, so3):
    sins = (si0, si1, si2, si3)
    souts = (so0, so1, so2, so3)
    wid = lax.axis_index("s") * _NC + lax.axis_index("c")
    col0 = wid * _BPW
    pltpu.sync_copy(tab_hbm, tab_v)

    def offs(g):
        f0 = (g // _NBC) * _FC
        b0 = col0 + (g % _NBC) * _BC
        return f0, b0

    def start_in(g, s):
        f0, b0 = offs(g)
        pltpu.async_copy(
            x_hbm.at[pl.ds(f0, _FC), :, pl.ds(b0, _BC)], bb.at[s], sins[s]
        )

    def wait_in(s):
        pltpu.make_async_copy(
            x_hbm.at[pl.ds(0, _FC), :, pl.ds(0, _BC)], bb.at[s], sins[s]
        ).wait()

    def start_out(g, s):
        f0, b0 = offs(g)
        pltpu.async_copy(
            bb.at[s], out_hbm.at[pl.ds(f0, _FC), :, pl.ds(b0, _BC)], souts[s]
        )

    def wait_out(s):
        pltpu.make_async_copy(
            bb.at[s], out_hbm.at[pl.ds(0, _FC), :, pl.ds(0, _BC)], souts[s]
        ).wait()

    def compute(g, s):
        f0 = (g // _NBC) * _FC
        for f in range(_FC):
            def dbody(d, c2):
                t = tab_v[0, pl.ds(((f0 + f) * _D + d) * _LANES, _LANES)]
                for k in range(_KB):
                    sl = pl.ds(k * _LANES, _LANES)
                    bb[s, f, d, sl] = bb[s, f, d, sl] + t
                return c2

            lax.fori_loop(0, _D, dbody, 0, unroll=2)

    def step(g, s, with_wait_out):
        wait_in(s)
        if with_wait_out:
            wait_out((s + 2) % 4)
        nxt = jnp.minimum(g + 2, _NCH - 1)
        start_in(nxt, (s + 2) % 4)
        compute(g, s)
        start_out(g, s)

    start_in(0, 0)
    start_in(1, 1)
    step(0, 0, with_wait_out=False)
    step(1, 1, with_wait_out=False)

    def quad(p, carry):
        for j in range(4):
            step(2 + p * 4 + j, (2 + j) % 4, with_wait_out=True)
        return carry

    lax.fori_loop(0, (_NCH - 4) // 4, quad, 0)
    step(_NCH - 2, (_NCH - 2) % 4, with_wait_out=True)
    step(_NCH - 1, (_NCH - 1) % 4, with_wait_out=True)

    wait_in(_NCH % 4)
    wait_in((_NCH + 1) % 4)
    wait_out((_NCH - 2) % 4)
    wait_out((_NCH - 1) % 4)


def kernel(inputs, column_table):
    xt = jnp.transpose(inputs, (1, 2, 0))
    tab = jnp.repeat(column_table.reshape(-1), _LANES).reshape(1, _ROW * _LANES)
    mesh = plsc.VectorSubcoreMesh(core_axis_name="c", subcore_axis_name="s")
    out_t = pl.kernel(
        _sc_body,
        out_type=jax.ShapeDtypeStruct((_F, _D, _BATCH), jnp.float32),
        mesh=mesh,
        scratch_types=[
            pltpu.VMEM((1, _ROW * _LANES), jnp.float32),
            pltpu.VMEM((4, _FC, _D, _BC), jnp.float32),
            pltpu.SemaphoreType.DMA,
            pltpu.SemaphoreType.DMA,
            pltpu.SemaphoreType.DMA,
            pltpu.SemaphoreType.DMA,
            pltpu.SemaphoreType.DMA,
            pltpu.SemaphoreType.DMA,
            pltpu.SemaphoreType.DMA,
            pltpu.SemaphoreType.DMA,
        ],
    )(xt, tab)
    return jnp.transpose(out_t, (2, 0, 1))
